# Initial kernel scaffold; baseline (speedup 1.0000x reference)
import jax
import jax.numpy as jnp
from jax.experimental import pallas as pl


def kernel(graphs, edge_index, params):
    raise NotImplementedError("write your pallas kernel here")


# speedup vs baseline: 4.1272x; 4.1272x over previous
import functools

import jax
import jax.numpy as jnp
from jax import lax
from jax.experimental import pallas as pl
from jax.experimental.pallas import tpu as pltpu
from jax.experimental.pallas import tpu_sc as plsc

N = 10000
E = 320000
T = 4
F_IN = 128
K = 2

NP = 10240
DUMMY = N
NCORES = 2
NSUB = 16
NTILES = NCORES * NSUB
EPT = E // NTILES
C = 80
NCH = EPT // C
RPT = NP // NSUB
ZR = 128

DIN = [128, 256, 128]
DOUT = [256, 128, 2]
DP = [256, 128, 128]
XW = [[128], [128, 128], [128]]
HW = [[128, 128], [128], [128]]

BN = 1000
GRID = N // BN

_f32 = jnp.float32
_i32 = jnp.int32


@functools.lru_cache(maxsize=None)
def _make_prop(widths):
    widths = tuple(widths)
    n_in = len(widths)
    uw = tuple(sorted(set(widths)))

    scratch = [
        pltpu.VMEM((2, C), _i32),
        pltpu.VMEM((2, C), _i32),
        pltpu.SemaphoreType.DMA,
        pltpu.SemaphoreType.DMA,
    ]
    for w in uw:
        scratch.append(pltpu.VMEM((2, C, w), _f32))
        scratch.append(pltpu.VMEM((ZR, w), _f32))
        scratch.append(pltpu.VMEM_SHARED((NP, w), _f32))

    out_type = [jax.ShapeDtypeStruct((NCORES, NP, w), _f32) for w in widths]
    mesh = plsc.VectorSubcoreMesh(
        core_axis_name="c", subcore_axis_name="s",
        num_cores=NCORES, num_subcores=NSUB)

    def body(*refs):
        row_h, col_h = refs[0], refs[1]
        xs = refs[2:2 + n_in]
        outs = refs[2 + n_in:2 + 2 * n_in]
        scr = refs[2 + 2 * n_in:]
        row_buf, col_buf, gsem0, gsem1 = scr[:4]
        per_w = {}
        for j, w in enumerate(uw):
            per_w[w] = (scr[4 + 3 * j], scr[5 + 3 * j], scr[6 + 3 * j])

        cid = lax.axis_index("c")
        sid = lax.axis_index("s")
        ebase = (cid * NSUB + sid) * EPT
        rbase = sid * RPT
        zero16 = jnp.zeros((16,), _f32)

        for w in uw:
            zbuf = per_w[w][1]

            def zrow(r, carry, _zbuf=zbuf, _w=w):
                for j in range(_w // 16):
                    _zbuf[r, pl.ds(j * 16, 16)] = zero16
                return carry

            lax.fori_loop(0, ZR, zrow, 0)

        for k, w in enumerate(widths):
            rows_buf, zbuf, accum = per_w[w]
            x_h = xs[k]
            out = outs[k]

            def zcp(m, carry, _zbuf=zbuf, _accum=accum):
                pltpu.sync_copy(_zbuf, _accum.at[pl.ds(rbase + m * ZR, ZR)])
                return carry

            lax.fori_loop(0, RPT // ZR, zcp, 0)
            plsc.subcore_barrier()

            def pair(i, carry, _rows=rows_buf, _x=x_h, _accum=accum):
                for b in (0, 1):
                    off = ebase + (i * 2 + b) * C
                    pltpu.sync_copy(row_h.at[pl.ds(off, C)], row_buf.at[b])
                    pltpu.sync_copy(col_h.at[pl.ds(off, C)], col_buf.at[b])
                d0 = pltpu.async_copy(_x.at[row_buf.at[0]], _rows.at[0], gsem0)
                d1 = pltpu.async_copy(_x.at[row_buf.at[1]], _rows.at[1], gsem1)
                d0.wait()
                pltpu.sync_copy(_rows.at[0], _accum.at[col_buf.at[0]], add=True)
                d1.wait()
                pltpu.sync_copy(_rows.at[1], _accum.at[col_buf.at[1]], add=True)
                return carry

            lax.fori_loop(0, NCH // 2, pair, 0)

            off = ebase + (NCH - 1) * C
            pltpu.sync_copy(row_h.at[pl.ds(off, C)], row_buf.at[0])
            pltpu.sync_copy(col_h.at[pl.ds(off, C)], col_buf.at[0])
            pltpu.async_copy(x_h.at[row_buf.at[0]], rows_buf.at[0], gsem0).wait()
            pltpu.sync_copy(rows_buf.at[0], accum.at[col_buf.at[0]], add=True)

            plsc.subcore_barrier()
            pltpu.sync_copy(accum.at[pl.ds(rbase, RPT)],
                            out.at[cid, pl.ds(rbase, RPT)])

    return pl.kernel(body, out_type=out_type, mesh=mesh,
                     scratch_types=scratch)


def _redirect(row, col):
    r2 = row.reshape(E // 128, 128)
    c2 = col.reshape(E // 128, 128)

    def body(r_ref, c_ref, col2_ref, rowd_ref):
        r = r_ref[...]
        c = c_ref[...]
        is_self = r == c
        col2_ref[...] = jnp.where(is_self, DUMMY, c)
        rowd_ref[...] = jnp.where(is_self, DUMMY, r)

    col2, rowd = pl.pallas_call(
        body,
        out_shape=[jax.ShapeDtypeStruct((E // 128, 128), _i32)] * 2,
    )(r2, c2)
    return col2.reshape(E), rowd.reshape(E)


def _prep(degp, graphs):
    def body(degp_ref, g_ref, dis_ref, gs_ref):
        deg = degp_ref[0, :, :16] + degp_ref[1, :, :16]
        dis = jnp.where(deg > 0, lax.rsqrt(deg), 0.0)
        dis_ref[...] = dis
        d1 = dis[:, :1]
        for t in range(T):
            gs_ref[t] = g_ref[t] * d1

    return pl.pallas_call(
        body,
        grid=(GRID,),
        in_specs=[
            pl.BlockSpec((NCORES, BN, 128), lambda i: (0, i, 0)),
            pl.BlockSpec((T, BN, F_IN), lambda i: (0, i, 0)),
        ],
        out_specs=[
            pl.BlockSpec((BN, 16), lambda i: (i, 0)),
            pl.BlockSpec((T, BN, F_IN), lambda i: (0, i, 0)),
        ],
        out_shape=[
            jax.ShapeDtypeStruct((N, 16), _f32),
            jax.ShapeDtypeStruct((T, N, F_IN), _f32),
        ],
    )(degp, graphs)


def _part_spec():
    return pl.BlockSpec((NCORES, BN, None), lambda i: (0, i, 0))


def _full(shape):
    nd = len(shape)
    return pl.BlockSpec(shape, lambda i: (0,) * nd)


def _gates(layer, t0, X, H, dis16, px_parts, ph_parts, wts):
    din = DIN[layer]
    dp = DP[layer]
    xw = XW[layer]
    hw = HW[layer]
    Wx0, Wx1, bx, Wh0, Wh1, bh, Whh0, bhh = wts[:8]
    nx = len(px_parts)
    nh = len(ph_parts)

    def body(*refs):
        i = 0
        x_ref = refs[i]; i += 1
        if not t0:
            h_ref = refs[i]; i += 1
        dis_ref = refs[i]; i += 1
        pxr = refs[i:i + nx]; i += nx
        phr = refs[i:i + nh] if not t0 else []
        i += len(phr)
        wx0_r, wx1_r, bx_r, wh0_r, wh1_r, bh_r, whh0_r, bhh_r = refs[i:i + 8]
        i += 8
        z_ref, a3_ref = refs[i], refs[i + 1]
        hrs_refs = refs[i + 2:]

        d1 = dis_ref[:, :1]
        nd = -d1
        ax = jnp.dot(x_ref[...], wx0_r[...], preferred_element_type=_f32)
        wx1 = wx1_r[...]
        off = 0
        for k, w in enumerate(xw):
            px = (pxr[k][0] + pxr[k][1]) * nd
            ax = ax + jnp.dot(px, wx1[off:off + w], preferred_element_type=_f32)
            off += w
        ax = ax + bx_r[...]

        if t0:
            ah = jnp.broadcast_to(bh_r[...], (BN, 2 * dp))
        else:
            h = h_ref[...]
            ah = jnp.dot(h, wh0_r[...], preferred_element_type=_f32)
            wh1 = wh1_r[...]
            off = 0
            for k, w in enumerate(hw):
                ph = (phr[k][0] + phr[k][1]) * nd
                ah = ah + jnp.dot(ph, wh1[off:off + w],
                                  preferred_element_type=_f32)
                off += w
            ah = ah + bh_r[...]

        z = jax.nn.sigmoid(ax[:, :dp] + ah[:, :dp])
        z_ref[...] = z
        if t0:
            a3_ref[...] = ax[:, 2 * dp:] + bhh_r[...]
        else:
            r = jax.nn.sigmoid(ax[:, dp:2 * dp] + ah[:, dp:2 * dp])
            hr = h * r
            a3_ref[...] = (ax[:, 2 * dp:] + bhh_r[...]
                           + jnp.dot(hr, whh0_r[...],
                                     preferred_element_type=_f32))
            off = 0
            for k, w in enumerate(hw):
                hrs_refs[k][...] = d1 * hr[:, off:off + w]
                off += w

    in_arrays = [X] + ([] if t0 else [H]) + [dis16] + list(px_parts) \
        + ([] if t0 else list(ph_parts)) + [Wx0, Wx1, bx, Wh0, Wh1, bh,
                                            Whh0, bhh]
    in_specs = [pl.BlockSpec((BN, din), lambda i: (i, 0))]
    if not t0:
        in_specs.append(pl.BlockSpec((BN, dp), lambda i: (i, 0)))
    in_specs.append(pl.BlockSpec((BN, 16), lambda i: (i, 0)))
    for w in xw:
        in_specs.append(pl.BlockSpec((NCORES, BN, w), lambda i: (0, i, 0)))
    if not t0:
        for w in hw:
            in_specs.append(pl.BlockSpec((NCORES, BN, w), lambda i: (0, i, 0)))
    for a in [Wx0, Wx1, bx, Wh0, Wh1, bh, Whh0, bhh]:
        in_specs.append(_full(a.shape))

    out_shape = [jax.ShapeDtypeStruct((N, dp), _f32),
                 jax.ShapeDtypeStruct((N, dp), _f32)]
    out_specs = [pl.BlockSpec((BN, dp), lambda i: (i, 0)),
                 pl.BlockSpec((BN, dp), lambda i: (i, 0))]
    if not t0:
        for w in hw:
            out_shape.append(jax.ShapeDtypeStruct((N, w), _f32))
            out_specs.append(pl.BlockSpec((BN, w), lambda i: (i, 0)))

    res = pl.pallas_call(
        body, grid=(GRID,), in_specs=in_specs,
        out_specs=out_specs, out_shape=out_shape,
    )(*in_arrays)
    if t0:
        return res[0], res[1], []
    return res[0], res[1], list(res[2:])


def _final(layer, t0, Z, A3p, H, dis16, phr_parts, Whh1):
    dp = DP[layer]
    hw = HW[layer]
    relu = layer > 0
    nh = len(phr_parts)

    def body(*refs):
        i = 0
        z_ref = refs[i]; i += 1
        a3_ref = refs[i]; i += 1
        if not t0:
            h_ref = refs[i]; i += 1
        dis_ref = refs[i]; i += 1
        phr = refs[i:i + nh]; i += nh
        if not t0:
            whh1_r = refs[i]; i += 1
        hn_ref = refs[i]
        hns_refs = refs[i + 1:]

        d1 = dis_ref[:, :1]
        ht_in = a3_ref[...]
        if not t0:
            whh1 = whh1_r[...]
            nd = -d1
            off = 0
            for k, w in enumerate(hw):
                p = (phr[k][0] + phr[k][1]) * nd
                ht_in = ht_in + jnp.dot(p, whh1[off:off + w],
                                        preferred_element_type=_f32)
                off += w
        ht = jnp.tanh(ht_in)
        z = z_ref[...]
        if t0:
            hn = (1.0 - z) * ht
        else:
            hn = z * h_ref[...] + (1.0 - z) * ht
        if relu:
            hn = jnp.maximum(hn, 0.0)
        hn_ref[...] = hn
        off = 0
        for k, w in enumerate(hw):
            hns_refs[k][...] = d1 * hn[:, off:off + w]
            off += w

    in_arrays = [Z, A3p] + ([] if t0 else [H]) + [dis16]
    in_specs = [pl.BlockSpec((BN, dp), lambda i: (i, 0)),
                pl.BlockSpec((BN, dp), lambda i: (i, 0))]
    if not t0:
        in_specs.append(pl.BlockSpec((BN, dp), lambda i: (i, 0)))
    in_specs.append(pl.BlockSpec((BN, 16), lambda i: (i, 0)))
    if not t0:
        in_arrays += list(phr_parts)
        for w in hw:
            in_specs.append(pl.BlockSpec((NCORES, BN, w), lambda i: (0, i, 0)))
        in_arrays.append(Whh1)
        in_specs.append(_full(Whh1.shape))

    out_shape = [jax.ShapeDtypeStruct((N, dp), _f32)]
    out_specs = [pl.BlockSpec((BN, dp), lambda i: (i, 0))]
    for w in hw:
        out_shape.append(jax.ShapeDtypeStruct((N, w), _f32))
        out_specs.append(pl.BlockSpec((BN, w), lambda i: (i, 0)))

    res = pl.pallas_call(
        body, grid=(GRID,), in_specs=in_specs,
        out_specs=out_specs, out_shape=out_shape,
    )(*in_arrays)
    return res[0], list(res[1:])


def _prep_weights(params):
    wts = []
    for layer, lp in enumerate(params):
        dout = DOUT[layer]
        dp = DP[layer]
        cpad = dp - dout

        def padw(w, rpad):
            return jnp.pad(w, ((0, rpad), (0, cpad)))

        Wx0 = jnp.concatenate(
            [padw(lp[g]['W'][0], 0) for g in ('x_z', 'x_r', 'x_h')], axis=1)
        Wx1 = jnp.concatenate(
            [padw(lp[g]['W'][1], 0) for g in ('x_z', 'x_r', 'x_h')], axis=1)
        bx = jnp.concatenate(
            [jnp.pad(lp[g]['b'], (0, cpad)) for g in ('x_z', 'x_r', 'x_h')]
        ).reshape(1, 3 * dp)
        Wh0 = jnp.concatenate(
            [padw(lp[g]['W'][0], cpad) for g in ('h_z', 'h_r')], axis=1)
        Wh1 = jnp.concatenate(
            [padw(lp[g]['W'][1], cpad) for g in ('h_z', 'h_r')], axis=1)
        bh = jnp.concatenate(
            [jnp.pad(lp[g]['b'], (0, cpad)) for g in ('h_z', 'h_r')]
        ).reshape(1, 2 * dp)
        Whh0 = padw(lp['h_h']['W'][0], cpad)
        Whh1 = padw(lp['h_h']['W'][1], cpad)
        bhh = jnp.pad(lp['h_h']['b'], (0, cpad)).reshape(1, dp)
        wts.append((Wx0, Wx1, bx, Wh0, Wh1, bh, Whh0, bhh, Whh1))
    return wts


def kernel(graphs, edge_index, params):
    row = edge_index[0]
    col = edge_index[1]
    col2, rowd = _redirect(row, col)

    ones128 = jnp.ones((N, 128), _f32)
    (degp,) = _make_prop((128,))(row, rowd, ones128)
    dis16, gs = _prep(degp, graphs)

    wts = _prep_weights(params)

    H = [None] * 3
    Hs = [None] * 3
    preds = []
    for t in range(T):
        x = graphs[t]
        xs_chunks = [gs[t]]
        for i in range(3):
            t0 = t == 0
            xw = XW[i]
            hw = HW[i]
            if t0:
                parts = _make_prop(tuple(xw))(row, col2, *xs_chunks)
                px = parts
                ph = []
            else:
                parts = _make_prop(tuple(xw + hw))(
                    row, col2, *(list(xs_chunks) + list(Hs[i])))
                px = parts[:len(xw)]
                ph = parts[len(xw):]
            z, a3p, hrs = _gates(i, t0, x, H[i], dis16, px, ph, wts[i])
            if t0:
                phr = []
            else:
                phr = _make_prop(tuple(hw))(row, col2, *hrs)
            hn, hns = _final(i, t0, z, a3p, H[i], dis16, phr, wts[i][8])
            H[i] = hn
            Hs[i] = hns
            x = hn
            xs_chunks = hns
        preds.append(H[2][:, :2])
    return jnp.stack(preds)
